# Initial kernel scaffold; baseline (speedup 1.0000x reference)
import jax
import jax.numpy as jnp
from jax.experimental import pallas as pl


def kernel(z, edge_index, pos_edge_index, eps):
    raise NotImplementedError("write your pallas kernel here")


# speedup vs baseline: 58.6144x; 58.6144x over previous
import functools

import jax
import jax.numpy as jnp
from jax import lax
from jax.experimental import pallas as pl
from jax.experimental.pallas import tpu as pltpu
from jax.experimental.pallas import tpu_sc as plsc

_NC = 2
_NS = 16
_L = 16


def _histogram_kernel(n_pad: int, e: int):
    eph = e // _NS
    sl = n_pad // _NS
    mesh = plsc.VectorSubcoreMesh(core_axis_name="c", subcore_axis_name="s")

    @functools.partial(
        pl.kernel,
        mesh=mesh,
        compiler_params=pltpu.CompilerParams(needs_layout_passes=False),
        out_type=jax.ShapeDtypeStruct((_NC * n_pad,), jnp.float32),
        scratch_types=[
            pltpu.VMEM((eph,), jnp.int32),
            pltpu.VMEM((n_pad,), jnp.int32),
            pltpu.VMEM((sl,), jnp.int32),
            pltpu.VMEM((sl,), jnp.float32),
            pltpu.VMEM((_L,), jnp.float32),
            pltpu.VMEM_SHARED((_NS * n_pad,), jnp.int32),
        ],
    )
    def hist(pos_hbm, eps_hbm, deg_hbm, idx_v, hist_v, buf_v, acc_v, eps_v,
             shared_v):
        c = lax.axis_index("c")
        s = lax.axis_index("s")
        row = 1 - c

        zero = jnp.zeros((_L,), jnp.int32)

        def zero_body(i, _):
            hist_v[pl.ds(i * _L, _L)] = zero
            return 0

        lax.fori_loop(0, n_pad // _L, zero_body, 0, unroll=4)

        src = pl.multiple_of(row * e + s * eph, 8)
        pltpu.sync_copy(pos_hbm.at[pl.ds(src, eph)], idx_v)
        pltpu.sync_copy(eps_hbm, eps_v)

        def hist_body(i, _):
            idx = idx_v[pl.ds(i * _L, _L)]
            cnt, last = plsc.scan_count(idx)
            plsc.addupdate_scatter(hist_v, [idx], cnt, mask=last)
            return 0

        lax.fori_loop(0, eph // _L, hist_body, 0)

        dst = pl.multiple_of(s * n_pad, 8)
        pltpu.sync_copy(hist_v, shared_v.at[pl.ds(dst, n_pad)])
        plsc.subcore_barrier()

        my = pl.multiple_of(s * sl, 8)
        epsv = eps_v[...]

        def init_body(j, _):
            acc_v[pl.ds(j * _L, _L)] = epsv
            return 0

        lax.fori_loop(0, sl // _L, init_body, 0, unroll=4)

        def red_tile(t, _):
            off = pl.multiple_of(t * n_pad + my, 8)
            pltpu.sync_copy(shared_v.at[pl.ds(off, sl)], buf_v)

            def add_body(j, _):
                d = pl.ds(j * _L, _L)
                acc_v[d] = acc_v[d] + buf_v[d].astype(jnp.float32)
                return 0

            lax.fori_loop(0, sl // _L, add_body, 0, unroll=4)
            return 0

        lax.fori_loop(0, _NS, red_tile, 0)

        out = pl.multiple_of(c * n_pad + my, 8)
        pltpu.sync_copy(acc_v, deg_hbm.at[pl.ds(out, sl)])

    return hist


def _gather_kernel(n_pad: int, e: int):
    epw = e // (_NC * _NS)
    mesh = plsc.VectorSubcoreMesh(core_axis_name="c", subcore_axis_name="s")

    @functools.partial(
        pl.kernel,
        mesh=mesh,
        compiler_params=pltpu.CompilerParams(needs_layout_passes=False),
        out_type=jax.ShapeDtypeStruct((e,), jnp.float32),
        scratch_types=[
            pltpu.VMEM((n_pad,), jnp.float32),
            pltpu.VMEM((n_pad,), jnp.float32),
            pltpu.VMEM((epw,), jnp.int32),
            pltpu.VMEM((epw,), jnp.int32),
            pltpu.VMEM((epw,), jnp.float32),
        ],
    )
    def gather(deg_hbm, edge_hbm, out_hbm, din_v, dout_v, tidx_v, hidx_v,
               out_v):
        c = lax.axis_index("c")
        s = lax.axis_index("s")
        wid = s * _NC + c
        base = pl.multiple_of(wid * epw, 8)

        pltpu.sync_copy(deg_hbm.at[pl.ds(0, n_pad)], din_v)
        pltpu.sync_copy(deg_hbm.at[pl.ds(n_pad, n_pad)], dout_v)
        pltpu.sync_copy(edge_hbm.at[pl.ds(e + base, epw)], tidx_v)
        pltpu.sync_copy(edge_hbm.at[pl.ds(base, epw)], hidx_v)

        @plsc.parallel_loop(0, epw // _L, unroll=4)
        def _(i):
            d = pl.ds(i * _L, _L)
            a = plsc.load_gather(din_v, [tidx_v[d]])
            b = plsc.load_gather(dout_v, [hidx_v[d]])
            out_v[d] = (2.0 * a * b) / (a + b)

        pltpu.sync_copy(out_v, out_hbm.at[pl.ds(base, epw)])

    return gather


def kernel(z, edge_index, pos_edge_index, eps):
    n = z.shape[0]
    e = edge_index.shape[1]
    n_pad = (n + _NS * _L - 1) // (_NS * _L) * (_NS * _L)
    eps_vec = jnp.full((_L,), eps, jnp.float32)
    pos_flat = pos_edge_index.reshape(-1)
    edge_flat = edge_index.reshape(-1)
    deg = _histogram_kernel(n_pad, e)(pos_flat, eps_vec)
    return _gather_kernel(n_pad, e)(deg, edge_flat)


# speedup vs baseline: 81.7741x; 1.3951x over previous
import functools

import jax
import jax.numpy as jnp
from jax import lax
from jax.experimental import pallas as pl
from jax.experimental.pallas import tpu as pltpu
from jax.experimental.pallas import tpu_sc as plsc

_NC = 2
_NS = 16
_L = 16


def _histogram_kernel(n_pad: int, e: int):
    eph = e // _NS
    sl = n_pad // _NS
    mesh = plsc.VectorSubcoreMesh(core_axis_name="c", subcore_axis_name="s")

    @functools.partial(
        pl.kernel,
        mesh=mesh,
        compiler_params=pltpu.CompilerParams(needs_layout_passes=False),
        out_type=jax.ShapeDtypeStruct((_NC * n_pad,), jnp.float32),
        scratch_types=[
            pltpu.VMEM((eph,), jnp.int32),
            pltpu.VMEM((n_pad,), jnp.int32),
            pltpu.VMEM((sl,), jnp.int32),
            pltpu.VMEM((sl,), jnp.float32),
            pltpu.VMEM((_L,), jnp.float32),
            pltpu.VMEM_SHARED((_NS * n_pad,), jnp.int32),
        ],
    )
    def hist(pos_hbm, eps_hbm, deg_hbm, idx_v, hist_v, buf_v, acc_v, eps_v,
             shared_v):
        c = lax.axis_index("c")
        s = lax.axis_index("s")
        row = 1 - c

        zero = jnp.zeros((_L,), jnp.int32)

        def zero_body(i, _):
            hist_v[pl.ds(i * _L, _L)] = zero
            return 0

        lax.fori_loop(0, n_pad // _L, zero_body, 0, unroll=4)

        src = pl.multiple_of(row * e + s * eph, 8)
        pltpu.sync_copy(pos_hbm.at[pl.ds(src, eph)], idx_v)
        pltpu.sync_copy(eps_hbm, eps_v)

        @plsc.parallel_loop(0, eph // _L, unroll=8)
        def _(i):
            idx = idx_v[pl.ds(i * _L, _L)]
            cnt, last = plsc.scan_count(idx)
            plsc.addupdate_scatter(hist_v, [idx], cnt, mask=last)

        dst = pl.multiple_of(s * n_pad, 8)
        pltpu.sync_copy(hist_v, shared_v.at[pl.ds(dst, n_pad)])
        plsc.subcore_barrier()

        my = pl.multiple_of(s * sl, 8)
        epsv = eps_v[...]

        def init_body(j, _):
            acc_v[pl.ds(j * _L, _L)] = epsv
            return 0

        lax.fori_loop(0, sl // _L, init_body, 0, unroll=4)

        def red_tile(t, _):
            off = pl.multiple_of(t * n_pad + my, 8)
            pltpu.sync_copy(shared_v.at[pl.ds(off, sl)], buf_v)

            def add_body(j, _):
                d = pl.ds(j * _L, _L)
                acc_v[d] = acc_v[d] + buf_v[d].astype(jnp.float32)
                return 0

            lax.fori_loop(0, sl // _L, add_body, 0, unroll=4)
            return 0

        lax.fori_loop(0, _NS, red_tile, 0)

        out = pl.multiple_of(c * n_pad + my, 8)
        pltpu.sync_copy(acc_v, deg_hbm.at[pl.ds(out, sl)])

    return hist


def _gather_kernel(n_pad: int, e: int):
    epw = e // (_NC * _NS)
    mesh = plsc.VectorSubcoreMesh(core_axis_name="c", subcore_axis_name="s")

    @functools.partial(
        pl.kernel,
        mesh=mesh,
        compiler_params=pltpu.CompilerParams(needs_layout_passes=False),
        out_type=jax.ShapeDtypeStruct((e,), jnp.float32),
        scratch_types=[
            pltpu.VMEM((n_pad,), jnp.float32),
            pltpu.VMEM((n_pad,), jnp.float32),
            pltpu.VMEM((epw,), jnp.int32),
            pltpu.VMEM((epw,), jnp.int32),
            pltpu.VMEM((epw,), jnp.float32),
        ],
    )
    def gather(deg_hbm, edge_hbm, out_hbm, din_v, dout_v, tidx_v, hidx_v,
               out_v):
        c = lax.axis_index("c")
        s = lax.axis_index("s")
        wid = s * _NC + c
        base = pl.multiple_of(wid * epw, 8)

        pltpu.sync_copy(deg_hbm.at[pl.ds(0, n_pad)], din_v)
        pltpu.sync_copy(deg_hbm.at[pl.ds(n_pad, n_pad)], dout_v)
        pltpu.sync_copy(edge_hbm.at[pl.ds(e + base, epw)], tidx_v)
        pltpu.sync_copy(edge_hbm.at[pl.ds(base, epw)], hidx_v)

        @plsc.parallel_loop(0, epw // _L, unroll=4)
        def _(i):
            d = pl.ds(i * _L, _L)
            a = plsc.load_gather(din_v, [tidx_v[d]])
            b = plsc.load_gather(dout_v, [hidx_v[d]])
            out_v[d] = (2.0 * a * b) / (a + b)

        pltpu.sync_copy(out_v, out_hbm.at[pl.ds(base, epw)])

    return gather


def kernel(z, edge_index, pos_edge_index, eps):
    n = z.shape[0]
    e = edge_index.shape[1]
    n_pad = (n + _NS * _L - 1) // (_NS * _L) * (_NS * _L)
    eps_vec = jnp.full((_L,), eps, jnp.float32)
    pos_flat = pos_edge_index.reshape(-1)
    edge_flat = edge_index.reshape(-1)
    deg = _histogram_kernel(n_pad, e)(pos_flat, eps_vec)
    return _gather_kernel(n_pad, e)(deg, edge_flat)


# speedup vs baseline: 82.4557x; 1.0083x over previous
import functools

import jax
import jax.numpy as jnp
from jax import lax
from jax.experimental import pallas as pl
from jax.experimental.pallas import tpu as pltpu
from jax.experimental.pallas import tpu_sc as plsc

_NC = 2
_NS = 16
_L = 16


def _histogram_kernel(n_pad: int, e: int):
    eph = e // _NS
    sl = n_pad // _NS
    mesh = plsc.VectorSubcoreMesh(core_axis_name="c", subcore_axis_name="s")

    @functools.partial(
        pl.kernel,
        mesh=mesh,
        compiler_params=pltpu.CompilerParams(needs_layout_passes=False),
        out_type=jax.ShapeDtypeStruct((_NC * n_pad,), jnp.float32),
        scratch_types=[
            pltpu.VMEM((eph,), jnp.int32),
            pltpu.VMEM((n_pad,), jnp.int32),
            pltpu.VMEM((sl,), jnp.int32),
            pltpu.VMEM((sl,), jnp.float32),
            pltpu.VMEM((_L,), jnp.float32),
            pltpu.VMEM_SHARED((_NS * n_pad,), jnp.int32),
        ],
    )
    def hist(pos_hbm, eps_hbm, deg_hbm, idx_v, hist_v, buf_v, acc_v, eps_v,
             shared_v):
        c = lax.axis_index("c")
        s = lax.axis_index("s")
        row = 1 - c

        zero = jnp.zeros((_L,), jnp.int32)

        def zero_body(i, _):
            hist_v[pl.ds(i * _L, _L)] = zero
            return 0

        lax.fori_loop(0, n_pad // _L, zero_body, 0, unroll=4)

        src = pl.multiple_of(row * e + s * eph, 8)
        pltpu.sync_copy(pos_hbm.at[pl.ds(src, eph)], idx_v)
        pltpu.sync_copy(eps_hbm, eps_v)

        ones = jnp.ones((_L,), jnp.int32)

        @plsc.parallel_loop(0, eph // _L, unroll=8)
        def _(i):
            idx = idx_v[pl.ds(i * _L, _L)]
            plsc.addupdate_scatter(hist_v, [idx], ones)

        dst = pl.multiple_of(s * n_pad, 8)
        pltpu.sync_copy(hist_v, shared_v.at[pl.ds(dst, n_pad)])
        plsc.subcore_barrier()

        my = pl.multiple_of(s * sl, 8)
        epsv = eps_v[...]

        def init_body(j, _):
            acc_v[pl.ds(j * _L, _L)] = epsv
            return 0

        lax.fori_loop(0, sl // _L, init_body, 0, unroll=4)

        def red_tile(t, _):
            off = pl.multiple_of(t * n_pad + my, 8)
            pltpu.sync_copy(shared_v.at[pl.ds(off, sl)], buf_v)

            def add_body(j, _):
                d = pl.ds(j * _L, _L)
                acc_v[d] = acc_v[d] + buf_v[d].astype(jnp.float32)
                return 0

            lax.fori_loop(0, sl // _L, add_body, 0, unroll=4)
            return 0

        lax.fori_loop(0, _NS, red_tile, 0)

        out = pl.multiple_of(c * n_pad + my, 8)
        pltpu.sync_copy(acc_v, deg_hbm.at[pl.ds(out, sl)])

    return hist


def _gather_kernel(n_pad: int, e: int):
    epw = e // (_NC * _NS)
    mesh = plsc.VectorSubcoreMesh(core_axis_name="c", subcore_axis_name="s")

    @functools.partial(
        pl.kernel,
        mesh=mesh,
        compiler_params=pltpu.CompilerParams(needs_layout_passes=False),
        out_type=jax.ShapeDtypeStruct((e,), jnp.float32),
        scratch_types=[
            pltpu.VMEM((n_pad,), jnp.float32),
            pltpu.VMEM((n_pad,), jnp.float32),
            pltpu.VMEM((epw,), jnp.int32),
            pltpu.VMEM((epw,), jnp.int32),
            pltpu.VMEM((epw,), jnp.float32),
        ],
    )
    def gather(deg_hbm, edge_hbm, out_hbm, din_v, dout_v, tidx_v, hidx_v,
               out_v):
        c = lax.axis_index("c")
        s = lax.axis_index("s")
        wid = s * _NC + c
        base = pl.multiple_of(wid * epw, 8)

        pltpu.sync_copy(deg_hbm.at[pl.ds(0, n_pad)], din_v)
        pltpu.sync_copy(deg_hbm.at[pl.ds(n_pad, n_pad)], dout_v)
        pltpu.sync_copy(edge_hbm.at[pl.ds(e + base, epw)], tidx_v)
        pltpu.sync_copy(edge_hbm.at[pl.ds(base, epw)], hidx_v)

        @plsc.parallel_loop(0, epw // _L, unroll=4)
        def _(i):
            d = pl.ds(i * _L, _L)
            a = plsc.load_gather(din_v, [tidx_v[d]])
            b = plsc.load_gather(dout_v, [hidx_v[d]])
            out_v[d] = (2.0 * a * b) / (a + b)

        pltpu.sync_copy(out_v, out_hbm.at[pl.ds(base, epw)])

    return gather


def kernel(z, edge_index, pos_edge_index, eps):
    n = z.shape[0]
    e = edge_index.shape[1]
    n_pad = (n + _NS * _L - 1) // (_NS * _L) * (_NS * _L)
    eps_vec = jnp.full((_L,), eps, jnp.float32)
    pos_flat = pos_edge_index.reshape(-1)
    edge_flat = edge_index.reshape(-1)
    deg = _histogram_kernel(n_pad, e)(pos_flat, eps_vec)
    return _gather_kernel(n_pad, e)(deg, edge_flat)


# speedup vs baseline: 93.5900x; 1.1350x over previous
import functools

import jax
import jax.numpy as jnp
from jax import lax
from jax.experimental import pallas as pl
from jax.experimental.pallas import tpu as pltpu
from jax.experimental.pallas import tpu_sc as plsc

_NC = 2
_NS = 16
_L = 16


def _histogram_kernel(n_pad: int, e: int):
    eph = e // _NS
    sl = n_pad // _NS
    mesh = plsc.VectorSubcoreMesh(core_axis_name="c", subcore_axis_name="s")

    @functools.partial(
        pl.kernel,
        mesh=mesh,
        compiler_params=pltpu.CompilerParams(needs_layout_passes=False),
        out_type=jax.ShapeDtypeStruct((_NC * n_pad,), jnp.float32),
        scratch_types=[
            pltpu.VMEM((eph,), jnp.int32),
            pltpu.VMEM((n_pad,), jnp.int32),
            pltpu.VMEM((_NS, sl), jnp.int32),
            pltpu.VMEM((sl,), jnp.float32),
            pltpu.VMEM((_L,), jnp.float32),
            pltpu.VMEM_SHARED((_NS * n_pad,), jnp.int32),
            pltpu.SemaphoreType.DMA,
        ],
    )
    def hist(pos_hbm, eps_hbm, deg_hbm, idx_v, hist_v, buf_v, acc_v, eps_v,
             shared_v, sem):
        c = lax.axis_index("c")
        s = lax.axis_index("s")
        row = 1 - c

        src = pl.multiple_of(row * e + s * eph, 8)
        idx_dma = pltpu.async_copy(pos_hbm.at[pl.ds(src, eph)], idx_v, sem)
        eps_dma = pltpu.async_copy(eps_hbm, eps_v, sem)

        zero = jnp.zeros((_L,), jnp.int32)

        def zero_body(i, _):
            hist_v[pl.ds(i * _L, _L)] = zero
            return 0

        lax.fori_loop(0, n_pad // _L, zero_body, 0, unroll=8)
        idx_dma.wait()
        eps_dma.wait()

        ones = jnp.ones((_L,), jnp.int32)

        @plsc.parallel_loop(0, eph // _L, unroll=8)
        def _(i):
            idx = idx_v[pl.ds(i * _L, _L)]
            plsc.addupdate_scatter(hist_v, [idx], ones)

        dst = pl.multiple_of(s * n_pad, 8)
        pltpu.sync_copy(hist_v, shared_v.at[pl.ds(dst, n_pad)])
        plsc.subcore_barrier()

        my = pl.multiple_of(s * sl, 8)
        dmas = []
        for t in range(_NS):
            off = pl.multiple_of(t * n_pad + my, 8)
            dmas.append(
                pltpu.async_copy(shared_v.at[pl.ds(off, sl)], buf_v.at[t],
                                 sem))
        for d in dmas:
            d.wait()

        epsv = eps_v[...]

        def red_body(j, _):
            d = pl.ds(j * _L, _L)
            acc = buf_v[0, d]
            for t in range(1, _NS):
                acc = acc + buf_v[t, d]
            acc_v[d] = acc.astype(jnp.float32) + epsv
            return 0

        lax.fori_loop(0, sl // _L, red_body, 0, unroll=4)

        out = pl.multiple_of(c * n_pad + my, 8)
        pltpu.sync_copy(acc_v, deg_hbm.at[pl.ds(out, sl)])

    return hist


def _gather_kernel(n_pad: int, e: int):
    epw = e // (_NC * _NS)
    mesh = plsc.VectorSubcoreMesh(core_axis_name="c", subcore_axis_name="s")

    @functools.partial(
        pl.kernel,
        mesh=mesh,
        compiler_params=pltpu.CompilerParams(needs_layout_passes=False),
        out_type=jax.ShapeDtypeStruct((e,), jnp.float32),
        scratch_types=[
            pltpu.VMEM((n_pad,), jnp.float32),
            pltpu.VMEM((n_pad,), jnp.float32),
            pltpu.VMEM((epw,), jnp.int32),
            pltpu.VMEM((epw,), jnp.int32),
            pltpu.VMEM((epw,), jnp.float32),
            pltpu.SemaphoreType.DMA,
        ],
    )
    def gather(deg_hbm, edge_hbm, out_hbm, din_v, dout_v, tidx_v, hidx_v,
               out_v, sem):
        c = lax.axis_index("c")
        s = lax.axis_index("s")
        wid = s * _NC + c
        base = pl.multiple_of(wid * epw, 8)

        dmas = [
            pltpu.async_copy(deg_hbm.at[pl.ds(0, n_pad)], din_v, sem),
            pltpu.async_copy(deg_hbm.at[pl.ds(n_pad, n_pad)], dout_v, sem),
            pltpu.async_copy(edge_hbm.at[pl.ds(e + base, epw)], tidx_v, sem),
            pltpu.async_copy(edge_hbm.at[pl.ds(base, epw)], hidx_v, sem),
        ]
        for d in dmas:
            d.wait()

        @plsc.parallel_loop(0, epw // _L, unroll=8)
        def _(i):
            d = pl.ds(i * _L, _L)
            a = plsc.load_gather(din_v, [tidx_v[d]])
            b = plsc.load_gather(dout_v, [hidx_v[d]])
            out_v[d] = (2.0 * a * b) / (a + b)

        pltpu.sync_copy(out_v, out_hbm.at[pl.ds(base, epw)])

    return gather


def kernel(z, edge_index, pos_edge_index, eps):
    n = z.shape[0]
    e = edge_index.shape[1]
    n_pad = (n + _NS * _L - 1) // (_NS * _L) * (_NS * _L)
    eps_vec = jnp.full((_L,), eps, jnp.float32)
    pos_flat = pos_edge_index.reshape(-1)
    edge_flat = edge_index.reshape(-1)
    deg = _histogram_kernel(n_pad, e)(pos_flat, eps_vec)
    return _gather_kernel(n_pad, e)(deg, edge_flat)
